# Initial kernel scaffold; baseline (speedup 1.0000x reference)
import jax
import jax.numpy as jnp
from jax.experimental import pallas as pl


def kernel(edge_index, emb, W1, b1, W2, b2):
    raise NotImplementedError("write your pallas kernel here")


# speedup vs baseline: 11.5778x; 11.5778x over previous
import functools

import jax
import jax.numpy as jnp
from jax import lax
from jax.experimental import pallas as pl
from jax.experimental.pallas import tpu as pltpu
from jax.experimental.pallas import tpu_sc as plsc

N_NODES = 50000
D = 128
N_EDGES = 800000

NPAD = 50176
E_PAD = 802816
NT = 16
NC = 2
ROWS_PER_TILE = NPAD // NT
Q = 196
NB_MAIN = 392
NG_MAIN = 8
NBG = NB_MAIN // NG_MAIN
NB_DEG = 196
NG_DEG = 2
NBG_DEG = NB_DEG // NG_DEG

_MESH = plsc.VectorSubcoreMesh(
    core_axis_name="c", subcore_axis_name="s", num_cores=NC, num_subcores=NT)
_f32 = jnp.float32


@functools.partial(
    pl.kernel,
    out_type=jax.ShapeDtypeStruct((NC, NPAD, 16), _f32),
    mesh=_MESH,
    compiler_params=pltpu.CompilerParams(use_tc_tiling_on_sc=False),
    scratch_types=[
        pltpu.VMEM_SHARED((NPAD, 16), _f32),
        pltpu.VMEM((NBG_DEG, 128), jnp.int32),
        pltpu.VMEM((128, 16), _f32),
        pltpu.VMEM((2 * Q, 16), _f32),
    ],
)
def _deg_kernel(dst3, ones_h, zeros_h, out, accum, dst_v, ones_v, dbuf):
    c = lax.axis_index("c")
    s = lax.axis_index("s")
    w = s * NC + c
    pltpu.sync_copy(ones_h, ones_v)
    pltpu.sync_copy(zeros_h, dbuf)
    for q in range(8):
        pltpu.sync_copy(
            dbuf, accum.at[pl.ds(s * ROWS_PER_TILE + q * 2 * Q, 2 * Q)])
    plsc.subcore_barrier()

    for g in range(NG_DEG):
        pltpu.sync_copy(dst3.at[w, pl.ds(g * NBG_DEG, NBG_DEG)], dst_v)

        def body(b, carry):
            pltpu.sync_copy(ones_v, accum.at[dst_v.at[b]], add=True)
            return carry

        lax.fori_loop(0, NBG_DEG, body, 0)
    plsc.subcore_barrier()
    for q in range(8):
        sl = pl.ds(s * ROWS_PER_TILE + q * 2 * Q, 2 * Q)
        pltpu.sync_copy(accum.at[sl], dbuf)
        pltpu.sync_copy(dbuf, out.at[c, sl])


@functools.partial(
    pl.kernel,
    out_type=tuple(jax.ShapeDtypeStruct((NPAD, 32), _f32) for _ in range(4)),
    mesh=_MESH,
    compiler_params=pltpu.CompilerParams(use_tc_tiling_on_sc=False),
    scratch_types=[
        pltpu.VMEM_SHARED((NPAD, 32), _f32),
        pltpu.VMEM((NBG, 128), jnp.int32),
        pltpu.VMEM((NBG, 128), jnp.int32),
        pltpu.VMEM((128, 32), _f32),
        pltpu.VMEM((Q, 32), _f32),
        pltpu.SemaphoreType.DMA,
    ],
)
def _scatter_kernel(h0, h1, h2, h3, src2, dst2, zeros_h,
                    o0, o1, o2, o3,
                    accum, src_v, dst_v, rows_v, zbuf, gsem):
    c = lax.axis_index("c")
    s = lax.axis_index("s")

    def chunk_pass(t_h, o_h):
        pltpu.sync_copy(zeros_h, zbuf)
        for q in range(16):
            pltpu.sync_copy(
                zbuf, accum.at[pl.ds(s * ROWS_PER_TILE + q * Q, Q)])
        plsc.subcore_barrier()
        for g in range(NG_MAIN):
            pltpu.sync_copy(src2.at[s, pl.ds(g * NBG, NBG)], src_v)
            pltpu.sync_copy(dst2.at[s, pl.ds(g * NBG, NBG)], dst_v)

            def body(b, carry):
                pltpu.async_copy(t_h.at[src_v.at[b]], rows_v, gsem).wait()
                pltpu.sync_copy(rows_v, accum.at[dst_v.at[b]], add=True)
                return carry

            lax.fori_loop(0, NBG, body, 0)
        plsc.subcore_barrier()
        for q in range(16):
            sl = pl.ds(s * ROWS_PER_TILE + q * Q, Q)
            pltpu.sync_copy(accum.at[sl], zbuf)
            pltpu.sync_copy(zbuf, o_h.at[sl])

    @pl.when(c == 0)
    def _():
        chunk_pass(h0, o0)
        chunk_pass(h1, o1)

    @pl.when(c == 1)
    def _():
        chunk_pass(h2, o2)
        chunk_pass(h3, o3)


_BLK = 512
_GRID = NPAD // _BLK


def _dinv_block(degp):
    dsum = degp[0] + degp[1]
    return lax.rsqrt(dsum[:, 0:1] + 1.0)


def _prescale_body(emb_ref, degp_ref, g0, g1, g2, g3):
    dinv = _dinv_block(degp_ref[...])
    hp = emb_ref[...] * dinv
    g0[...] = hp[:, 0:32]
    g1[...] = hp[:, 32:64]
    g2[...] = hp[:, 64:96]
    g3[...] = hp[:, 96:128]


def _tc_prescale(emb_pad, degp):
    chunk = pl.BlockSpec((_BLK, 32), lambda i: (i, 0))
    return pl.pallas_call(
        _prescale_body,
        grid=(_GRID,),
        in_specs=[
            pl.BlockSpec((_BLK, D), lambda i: (i, 0)),
            pl.BlockSpec((NC, _BLK, 16), lambda i: (0, i, 0)),
        ],
        out_specs=[chunk] * 4,
        out_shape=[jax.ShapeDtypeStruct((NPAD, 32), _f32)] * 4,
    )(emb_pad, degp)


def _mid_body(s0, s1, s2, s3, g0, g1, g2, g3, degp_ref, w1, b1, w2,
              f0, f1, f2, f3):
    dinv = _dinv_block(degp_ref[...])
    acc = jnp.concatenate([s0[...], s1[...], s2[...], s3[...]], axis=1)
    hp1 = jnp.concatenate([g0[...], g1[...], g2[...], g3[...]], axis=1)
    p1 = dinv * (acc + hp1)
    z = jnp.maximum(
        jnp.dot(p1, w1[...], preferred_element_type=_f32) + b1[...], 0.0)
    h2 = jnp.dot(z, w2[...], preferred_element_type=_f32)
    hp2 = h2 * dinv
    f0[...] = hp2[:, 0:32]
    f1[...] = hp2[:, 32:64]
    f2[...] = hp2[:, 64:96]
    f3[...] = hp2[:, 96:128]


def _tc_mid(ss, gs, degp, W1, b1, W2):
    chunk = pl.BlockSpec((_BLK, 32), lambda i: (i, 0))
    return pl.pallas_call(
        _mid_body,
        grid=(_GRID,),
        in_specs=[chunk] * 8 + [
            pl.BlockSpec((NC, _BLK, 16), lambda i: (0, i, 0)),
            pl.BlockSpec((D, 2 * D), lambda i: (0, 0)),
            pl.BlockSpec((1, 2 * D), lambda i: (0, 0)),
            pl.BlockSpec((2 * D, D), lambda i: (0, 0)),
        ],
        out_specs=[chunk] * 4,
        out_shape=[jax.ShapeDtypeStruct((NPAD, 32), _f32)] * 4,
    )(*ss, *gs, degp, W1, b1.reshape(1, 2 * D), W2)


def _final_body(s0, s1, s2, s3, f0, f1, f2, f3, degp_ref, b2, out):
    dinv = _dinv_block(degp_ref[...])
    acc = jnp.concatenate([s0[...], s1[...], s2[...], s3[...]], axis=1)
    hp2 = jnp.concatenate([f0[...], f1[...], f2[...], f3[...]], axis=1)
    out[...] = dinv * (acc + hp2) + b2[...]


def _tc_final(ss, fs, degp, b2):
    chunk = pl.BlockSpec((_BLK, 32), lambda i: (i, 0))
    return pl.pallas_call(
        _final_body,
        grid=(_GRID,),
        in_specs=[chunk] * 8 + [
            pl.BlockSpec((NC, _BLK, 16), lambda i: (0, i, 0)),
            pl.BlockSpec((1, D), lambda i: (0, 0)),
        ],
        out_specs=pl.BlockSpec((_BLK, D), lambda i: (i, 0)),
        out_shape=jax.ShapeDtypeStruct((N_NODES, D), _f32),
    )(*ss, *fs, degp, b2.reshape(1, D))


def kernel(edge_index, emb, W1, b1, W2, b2):
    src = edge_index[0].astype(jnp.int32)
    dst = edge_index[1].astype(jnp.int32)
    npad_extra = E_PAD - N_EDGES
    pad_ids = jnp.arange(npad_extra, dtype=jnp.int32)
    src_p = jnp.concatenate([src, pad_ids % N_NODES])
    dst_p = jnp.concatenate([dst, N_NODES + pad_ids % (NPAD - N_NODES)])
    src2 = src_p.reshape(NT, NB_MAIN, 128)
    dst2 = dst_p.reshape(NT, NB_MAIN, 128)
    dst3 = dst_p.reshape(NT * NC, NB_DEG, 128)

    emb_pad = jnp.pad(emb, ((0, NPAD - N_NODES), (0, 0)))
    ones16 = jnp.ones((128, 16), _f32)
    z16 = jnp.zeros((2 * Q, 16), _f32)
    z32 = jnp.zeros((Q, 32), _f32)

    degp = _deg_kernel(dst3, ones16, z16)
    gs = _tc_prescale(emb_pad, degp)
    ss = _scatter_kernel(*gs, src2, dst2, z32)
    fs = _tc_mid(ss, gs, degp, W1, b1, W2)
    ts = _scatter_kernel(*fs, src2, dst2, z32)
    return _tc_final(ts, fs, degp, b2)


# speedup vs baseline: 18.5490x; 1.6021x over previous
import functools

import jax
import jax.numpy as jnp
from jax import lax
from jax.experimental import pallas as pl
from jax.experimental.pallas import tpu as pltpu
from jax.experimental.pallas import tpu_sc as plsc

N_NODES = 50000
D = 128
N_EDGES = 800000

NPAD = 50176
E_PAD = 802816
NT = 16
NC = 2
ROWS_PER_TILE = NPAD // NT
Q = 196
NB_MAIN = 392
NG_MAIN = 14
NBG = NB_MAIN // NG_MAIN
UN2 = NBG // 4
NB_DEG = 196
NG_DEG = 2
NBG_DEG = NB_DEG // NG_DEG

_MESH = plsc.VectorSubcoreMesh(
    core_axis_name="c", subcore_axis_name="s", num_cores=NC, num_subcores=NT)
_f32 = jnp.float32


@functools.partial(
    pl.kernel,
    out_type=jax.ShapeDtypeStruct((NC, NPAD, 16), _f32),
    mesh=_MESH,
    compiler_params=pltpu.CompilerParams(use_tc_tiling_on_sc=False),
    scratch_types=[
        pltpu.VMEM_SHARED((NPAD, 16), _f32),
        pltpu.VMEM((NBG_DEG, 128), jnp.int32),
        pltpu.VMEM((128, 16), _f32),
    ],
)
def _deg_kernel(dst3, ones_h, zeros_h, out, accum, dst_v, ones_v):
    c = lax.axis_index("c")
    s = lax.axis_index("s")
    w = s * NC + c
    tsl = pl.ds(s * ROWS_PER_TILE, ROWS_PER_TILE)
    pltpu.sync_copy(ones_h, ones_v)
    pltpu.sync_copy(zeros_h, accum.at[tsl])
    plsc.subcore_barrier()

    for g in range(NG_DEG):
        pltpu.sync_copy(dst3.at[w, pl.ds(g * NBG_DEG, NBG_DEG)], dst_v)

        def body(b, carry):
            pltpu.sync_copy(ones_v, accum.at[dst_v.at[b]], add=True)
            return carry

        lax.fori_loop(0, NBG_DEG, body, 0)
    plsc.subcore_barrier()
    pltpu.sync_copy(accum.at[tsl], out.at[c, tsl])


@functools.partial(
    pl.kernel,
    out_type=jax.ShapeDtypeStruct((4, NPAD, 32), _f32),
    mesh=_MESH,
    compiler_params=pltpu.CompilerParams(use_tc_tiling_on_sc=False),
    scratch_types=[
        pltpu.VMEM_SHARED((NPAD, 32), _f32),
        pltpu.VMEM((NBG, 128), jnp.int32),
        pltpu.VMEM((NBG, 128), jnp.int32),
        pltpu.VMEM((128, 32), _f32),
        pltpu.VMEM((128, 32), _f32),
        pltpu.VMEM((128, 32), _f32),
        pltpu.VMEM((128, 32), _f32),
        pltpu.SemaphoreType.DMA,
        pltpu.SemaphoreType.DMA,
    ],
)
def _scatter_kernel(h4, src2, dst2, zeros_h, o4,
                    accum, src_v, dst_v, ra0, ra1, rb0, rb1, semA, semB):
    c = lax.axis_index("c")
    s = lax.axis_index("s")
    tsl = pl.ds(s * ROWS_PER_TILE, ROWS_PER_TILE)

    for kk in range(2):
        chunk = 2 * c + kk
        t_h = h4.at[chunk]
        pltpu.sync_copy(zeros_h, accum.at[tsl])
        plsc.subcore_barrier()

        def group(g, carry):
            pltpu.sync_copy(src2.at[s, pl.ds(g * NBG, NBG)], src_v)
            pltpu.sync_copy(dst2.at[s, pl.ds(g * NBG, NBG)], dst_v)

            def fire(u, bufs, sem):
                for t in range(2):
                    pltpu.async_copy(t_h.at[src_v.at[2 * u + t]], bufs[t],
                                     sem)

            def drain(u, bufs, sem):
                for t in range(2):
                    pltpu.make_async_copy(
                        t_h.at[src_v.at[2 * u + t]], bufs[t], sem).wait()

            def scat(u, bufs):
                for t in range(2):
                    pltpu.sync_copy(bufs[t], accum.at[dst_v.at[2 * u + t]],
                                    add=True)

            A = (ra0, ra1)
            B = (rb0, rb1)
            fire(0, A, semA)

            def body(u2, carry2):
                ua = 2 * u2
                ub = 2 * u2 + 1
                fire(ub, B, semB)
                drain(ua, A, semA)
                scat(ua, A)

                @pl.when(u2 < UN2 - 1)
                def _():
                    fire(ua + 2, A, semA)

                drain(ub, B, semB)
                scat(ub, B)
                return carry2

            lax.fori_loop(0, UN2, body, 0)
            return carry

        lax.fori_loop(0, NG_MAIN, group, 0)
        plsc.subcore_barrier()
        pltpu.sync_copy(accum.at[tsl], o4.at[chunk, tsl])


_BLK = 512
_GRID = NPAD // _BLK


def _dinv_block(degp):
    dsum = degp[0] + degp[1]
    return lax.rsqrt(dsum[:, 0:1] + 1.0)


_CHUNK4 = pl.BlockSpec((4, _BLK, 32), lambda i: (0, i, 0))
_CHUNK4_SHAPE = jax.ShapeDtypeStruct((4, NPAD, 32), _f32)


def _to_chunks(hp):
    return jnp.stack([hp[:, 32 * k:32 * k + 32] for k in range(4)])


def _from_chunks(x4):
    return jnp.concatenate([x4[k] for k in range(4)], axis=1)


def _prescale_body(emb_ref, degp_ref, g4):
    dinv = _dinv_block(degp_ref[...])
    g4[...] = _to_chunks(emb_ref[...] * dinv)


def _tc_prescale(emb_pad, degp):
    return pl.pallas_call(
        _prescale_body,
        grid=(_GRID,),
        in_specs=[
            pl.BlockSpec((_BLK, D), lambda i: (i, 0)),
            pl.BlockSpec((NC, _BLK, 16), lambda i: (0, i, 0)),
        ],
        out_specs=_CHUNK4,
        out_shape=_CHUNK4_SHAPE,
    )(emb_pad, degp)


def _mid_body(s4, g4, degp_ref, w1, b1, w2, f4):
    dinv = _dinv_block(degp_ref[...])
    p1 = dinv * (_from_chunks(s4[...]) + _from_chunks(g4[...]))
    z = jnp.maximum(
        jnp.dot(p1, w1[...], preferred_element_type=_f32) + b1[...], 0.0)
    h2 = jnp.dot(z, w2[...], preferred_element_type=_f32)
    f4[...] = _to_chunks(h2 * dinv)


def _tc_mid(s4, g4, degp, W1, b1, W2):
    return pl.pallas_call(
        _mid_body,
        grid=(_GRID,),
        in_specs=[_CHUNK4, _CHUNK4] + [
            pl.BlockSpec((NC, _BLK, 16), lambda i: (0, i, 0)),
            pl.BlockSpec((D, 2 * D), lambda i: (0, 0)),
            pl.BlockSpec((1, 2 * D), lambda i: (0, 0)),
            pl.BlockSpec((2 * D, D), lambda i: (0, 0)),
        ],
        out_specs=_CHUNK4,
        out_shape=_CHUNK4_SHAPE,
    )(s4, g4, degp, W1, b1.reshape(1, 2 * D), W2)


def _final_body(s4, f4, degp_ref, b2, out):
    dinv = _dinv_block(degp_ref[...])
    out[...] = dinv * (_from_chunks(s4[...]) + _from_chunks(f4[...])) + b2[...]


def _tc_final(s4, f4, degp, b2):
    return pl.pallas_call(
        _final_body,
        grid=(_GRID,),
        in_specs=[_CHUNK4, _CHUNK4] + [
            pl.BlockSpec((NC, _BLK, 16), lambda i: (0, i, 0)),
            pl.BlockSpec((1, D), lambda i: (0, 0)),
        ],
        out_specs=pl.BlockSpec((_BLK, D), lambda i: (i, 0)),
        out_shape=jax.ShapeDtypeStruct((N_NODES, D), _f32),
    )(s4, f4, degp, b2.reshape(1, D))


def kernel(edge_index, emb, W1, b1, W2, b2):
    src = edge_index[0].astype(jnp.int32)
    dst = edge_index[1].astype(jnp.int32)
    npad_extra = E_PAD - N_EDGES
    pad_ids = jnp.arange(npad_extra, dtype=jnp.int32)
    src_p = jnp.concatenate([src, pad_ids % N_NODES])
    dst_p = jnp.concatenate([dst, N_NODES + pad_ids % (NPAD - N_NODES)])
    src2 = src_p.reshape(NT, NB_MAIN, 128)
    dst2 = dst_p.reshape(NT, NB_MAIN, 128)
    dst3 = dst_p.reshape(NT * NC, NB_DEG, 128)

    emb_pad = jnp.pad(emb, ((0, NPAD - N_NODES), (0, 0)))
    ones16 = jnp.ones((128, 16), _f32)
    z16 = jnp.zeros((ROWS_PER_TILE, 16), _f32)
    z32 = jnp.zeros((ROWS_PER_TILE, 32), _f32)

    degp = _deg_kernel(dst3, ones16, z16)
    g4 = _tc_prescale(emb_pad, degp)
    s4 = _scatter_kernel(g4, src2, dst2, z32)
    f4 = _tc_mid(s4, g4, degp, W1, b1, W2)
    t4 = _scatter_kernel(f4, src2, dst2, z32)
    return _tc_final(t4, f4, degp, b2)


# speedup vs baseline: 19.8940x; 1.0725x over previous
import functools

import jax
import jax.numpy as jnp
from jax import lax
from jax.experimental import pallas as pl
from jax.experimental.pallas import tpu as pltpu
from jax.experimental.pallas import tpu_sc as plsc

N_NODES = 50000
D = 128
N_EDGES = 800000

NPAD = 50176
E_PAD = 802816
NT = 16
NC = 2
ROWS_PER_TILE = NPAD // NT
Q = 196
NB_MAIN = 392
NG_MAIN = 14
NBG = NB_MAIN // NG_MAIN
UN2 = NBG // 4
NB_DEG = 196
NG_DEG = 2
NBG_DEG = NB_DEG // NG_DEG

_MESH = plsc.VectorSubcoreMesh(
    core_axis_name="c", subcore_axis_name="s", num_cores=NC, num_subcores=NT)
_f32 = jnp.float32


@functools.partial(
    pl.kernel,
    out_type=jax.ShapeDtypeStruct((NC, NPAD, 16), _f32),
    mesh=_MESH,
    compiler_params=pltpu.CompilerParams(use_tc_tiling_on_sc=False),
    scratch_types=[
        pltpu.VMEM_SHARED((NPAD, 16), _f32),
        pltpu.VMEM((NBG_DEG, 128), jnp.int32),
        pltpu.VMEM((128, 16), _f32),
    ],
)
def _deg_kernel(dst3, ones_h, zeros_h, out, accum, dst_v, ones_v):
    c = lax.axis_index("c")
    s = lax.axis_index("s")
    w = s * NC + c
    tsl = pl.ds(s * ROWS_PER_TILE, ROWS_PER_TILE)
    pltpu.sync_copy(ones_h, ones_v)
    pltpu.sync_copy(zeros_h, accum.at[tsl])
    plsc.subcore_barrier()

    for g in range(NG_DEG):
        pltpu.sync_copy(dst3.at[w, pl.ds(g * NBG_DEG, NBG_DEG)], dst_v)

        def body(b, carry):
            pltpu.sync_copy(ones_v, accum.at[dst_v.at[b]], add=True)
            return carry

        lax.fori_loop(0, NBG_DEG, body, 0)
    plsc.subcore_barrier()
    pltpu.sync_copy(accum.at[tsl], out.at[c, tsl])


@functools.partial(
    pl.kernel,
    out_type=jax.ShapeDtypeStruct((NPAD, 4, 32), _f32),
    mesh=_MESH,
    compiler_params=pltpu.CompilerParams(use_tc_tiling_on_sc=False),
    scratch_types=[
        pltpu.VMEM_SHARED((NPAD, 32), _f32),
        pltpu.VMEM((NBG, 128), jnp.int32),
        pltpu.VMEM((NBG, 128), jnp.int32),
        pltpu.VMEM((128, 32), _f32),
        pltpu.VMEM((128, 32), _f32),
        pltpu.VMEM((128, 32), _f32),
        pltpu.VMEM((128, 32), _f32),
        pltpu.SemaphoreType.DMA,
        pltpu.SemaphoreType.DMA,
    ],
)
def _scatter_kernel(t_h, src24, dst2, zeros_h, o4,
                    accum, src_v, dst_v, ra0, ra1, rb0, rb1, semA, semB):
    c = lax.axis_index("c")
    s = lax.axis_index("s")
    tsl = pl.ds(s * ROWS_PER_TILE, ROWS_PER_TILE)

    for kk in range(2):
        chunk = 2 * c + kk
        pltpu.sync_copy(zeros_h, accum.at[tsl])
        plsc.subcore_barrier()

        def group(g, carry):
            pltpu.sync_copy(src24.at[chunk, s, pl.ds(g * NBG, NBG)], src_v)
            pltpu.sync_copy(dst2.at[s, pl.ds(g * NBG, NBG)], dst_v)

            def fire(u, bufs, sem):
                for t in range(2):
                    pltpu.async_copy(t_h.at[src_v.at[2 * u + t]], bufs[t],
                                     sem)

            def drain(u, bufs, sem):
                for t in range(2):
                    pltpu.make_async_copy(
                        t_h.at[src_v.at[2 * u + t]], bufs[t], sem).wait()

            def scat(u, bufs):
                for t in range(2):
                    pltpu.sync_copy(bufs[t], accum.at[dst_v.at[2 * u + t]],
                                    add=True)

            A = (ra0, ra1)
            B = (rb0, rb1)
            fire(0, A, semA)

            def body(u2, carry2):
                ua = 2 * u2
                ub = 2 * u2 + 1
                fire(ub, B, semB)
                drain(ua, A, semA)
                scat(ua, A)

                @pl.when(u2 < UN2 - 1)
                def _():
                    fire(ua + 2, A, semA)

                drain(ub, B, semB)
                scat(ub, B)
                return carry2

            lax.fori_loop(0, UN2, body, 0)
            return carry

        lax.fori_loop(0, NG_MAIN, group, 0)
        plsc.subcore_barrier()
        pltpu.sync_copy(accum.at[tsl], o4.at[tsl, chunk])


_BLK = 512
_GRID = NPAD // _BLK


def _dinv_block(degp):
    dsum = degp[0] + degp[1]
    return lax.rsqrt(dsum[:, 0:1] + 1.0)


_NAT = pl.BlockSpec((_BLK, D), lambda i: (i, 0))
_NAT_SHAPE = jax.ShapeDtypeStruct((NPAD, D), _f32)
_DEGP = pl.BlockSpec((NC, _BLK, 16), lambda i: (0, i, 0))


def _prescale_body(emb_ref, degp_ref, g):
    dinv = _dinv_block(degp_ref[...])
    g[...] = emb_ref[...] * dinv


def _tc_prescale(emb_pad, degp):
    return pl.pallas_call(
        _prescale_body,
        grid=(_GRID,),
        in_specs=[_NAT, _DEGP],
        out_specs=_NAT,
        out_shape=_NAT_SHAPE,
    )(emb_pad, degp)


def _mid_body(s, g, degp_ref, w1, b1, w2, f):
    dinv = _dinv_block(degp_ref[...])
    p1 = dinv * (s[...] + g[...])
    z = jnp.maximum(
        jnp.dot(p1, w1[...], preferred_element_type=_f32) + b1[...], 0.0)
    h2 = jnp.dot(z, w2[...], preferred_element_type=_f32)
    f[...] = h2 * dinv


def _tc_mid(s, g, degp, W1, b1, W2):
    return pl.pallas_call(
        _mid_body,
        grid=(_GRID,),
        in_specs=[_NAT, _NAT, _DEGP,
                  pl.BlockSpec((D, 2 * D), lambda i: (0, 0)),
                  pl.BlockSpec((1, 2 * D), lambda i: (0, 0)),
                  pl.BlockSpec((2 * D, D), lambda i: (0, 0))],
        out_specs=_NAT,
        out_shape=_NAT_SHAPE,
    )(s, g, degp, W1, b1.reshape(1, 2 * D), W2)


def _final_body(s, f, degp_ref, b2, out):
    dinv = _dinv_block(degp_ref[...])
    out[...] = dinv * (s[...] + f[...]) + b2[...]


def _tc_final(s, f, degp, b2):
    return pl.pallas_call(
        _final_body,
        grid=(_GRID,),
        in_specs=[_NAT, _NAT, _DEGP,
                  pl.BlockSpec((1, D), lambda i: (0, 0))],
        out_specs=pl.BlockSpec((_BLK, D), lambda i: (i, 0)),
        out_shape=jax.ShapeDtypeStruct((N_NODES, D), _f32),
    )(s, f, degp, b2.reshape(1, D))


def kernel(edge_index, emb, W1, b1, W2, b2):
    src = edge_index[0].astype(jnp.int32)
    dst = edge_index[1].astype(jnp.int32)
    npad_extra = E_PAD - N_EDGES
    pad_ids = jnp.arange(npad_extra, dtype=jnp.int32)
    src_p = jnp.concatenate([src, pad_ids % N_NODES])
    dst_p = jnp.concatenate([dst, N_NODES + pad_ids % (NPAD - N_NODES)])
    src24 = (4 * src_p[None, :] + jnp.arange(4, dtype=jnp.int32)[:, None]
             ).reshape(4, NT, NB_MAIN, 128)
    dst2 = dst_p.reshape(NT, NB_MAIN, 128)
    dst3 = dst_p.reshape(NT * NC, NB_DEG, 128)

    emb_pad = jnp.pad(emb, ((0, NPAD - N_NODES), (0, 0)))
    ones16 = jnp.ones((128, 16), _f32)
    z16 = jnp.zeros((ROWS_PER_TILE, 16), _f32)
    z32 = jnp.zeros((ROWS_PER_TILE, 32), _f32)

    degp = _deg_kernel(dst3, ones16, z16)
    g = _tc_prescale(emb_pad, degp)
    s = _scatter_kernel(g.reshape(4 * NPAD, 32), src24, dst2, z32)
    f = _tc_mid(s.reshape(NPAD, D), g, degp, W1, b1, W2)
    t = _scatter_kernel(f.reshape(4 * NPAD, 32), src24, dst2, z32)
    return _tc_final(t.reshape(NPAD, D), f, degp, b2)
